# baseline (speedup 1.0000x reference)
import functools

import jax
import jax.numpy as jnp
from jax import lax
from jax.experimental import pallas as pl
from jax.experimental.pallas import tpu as pltpu
from jax.experimental.pallas import tpu_sc as plsc

_NUM_CORES = 2
_NUM_SUBCORES = 16


def _tc_broadcast(W, n_batch, S, D):
    BS = 256

    def body(w_ref, o_ref):
        w = w_ref[...]
        for b in range(n_batch):
            o_ref[b] = w

    return pl.pallas_call(
        body,
        grid=(S // BS,),
        in_specs=[pl.BlockSpec((BS, D), lambda i: (i, 0))],
        out_specs=pl.BlockSpec((n_batch, BS, D), lambda i: (0, i, 0)),
        out_shape=jax.ShapeDtypeStruct((n_batch, S, D), jnp.float32),
    )(W)


def _sc_copy_one(W, S, D):
    NW = _NUM_CORES * _NUM_SUBCORES
    rows_per_w = S // NW
    CH = 64
    n_ch = rows_per_w // CH

    mesh = plsc.VectorSubcoreMesh(
        core_axis_name="c", subcore_axis_name="s", num_cores=_NUM_CORES
    )

    @functools.partial(
        pl.kernel,
        out_type=jax.ShapeDtypeStruct((1, S, D), jnp.float32),
        mesh=mesh,
        scratch_types=[pltpu.VMEM((CH, D), jnp.float32)],
    )
    def sc_copy(w_hbm, out_hbm, buf):
        wid = lax.axis_index("s") * _NUM_CORES + lax.axis_index("c")
        base0 = wid * rows_per_w
        for k in range(n_ch):
            base = base0 + k * CH
            pltpu.sync_copy(w_hbm.at[pl.ds(base, CH)], buf)
            pltpu.sync_copy(buf, out_hbm.at[0, pl.ds(base, CH)])

    return sc_copy(W)


def kernel(x, W):
    B, S = x.shape
    _, D = W.shape
    tc_part = _tc_broadcast(W, B - 1, S, D)
    sc_part = _sc_copy_one(W, S, D)
    return jnp.concatenate([tc_part, sc_part], axis=0)
